# baseline (speedup 1.0000x reference)
import functools

import jax
import jax.numpy as jnp
from jax import lax
from jax.experimental import pallas as pl
from jax.experimental.pallas import tpu as pltpu
from jax.experimental.pallas import tpu_sc as plsc

_NUM_NUMERICAL = 6
_N_CAT = 26
_VOCAB = 100000
_D = 64
_T = 20
_F = _NUM_NUMERICAL + _N_CAT
_ROWS = 1 + _T * _F
_NC, _NS, _L = 2, 16, 16
_NW = _NC * _NS


def _tokenizer_body(x_hbm, cls_hbm, w_hbm, b_hbm, table_hbm, pos_hbm, out_hbm,
                    xv, idx, block, posv, wv, bv, clsv, sem, b_per_w):
    wid = lax.axis_index("s") * _NC + lax.axis_index("c")
    base = wid * b_per_w

    pltpu.sync_copy(pos_hbm, posv)
    pltpu.sync_copy(w_hbm, wv)
    pltpu.sync_copy(b_hbm, bv)
    pltpu.sync_copy(cls_hbm, clsv)

    lane = lax.iota(jnp.int32, _L)

    def per_element(e, _):
        b = base + e
        pltpu.sync_copy(x_hbm.at[pl.ds(b * (_T * _F), _T * _F)], xv)

        def mk_idx(u, _):
            v = xv[pl.ds(u * _L, _L)]
            f = lane + (u % 2) * _L
            off = jnp.maximum(f - _NUM_NUMERICAL, 0) * _VOCAB
            iv = jnp.clip(v.astype(jnp.int32) + off, 0, _N_CAT * _VOCAB - 1)
            idx[u // 8, pl.ds((u % 8) * _L, _L)] = iv
            return _
        lax.fori_loop(0, (_T * _F) // _L, mk_idx, None)

        copies = [
            pltpu.make_async_copy(
                table_hbm.at[idx.at[k]],
                block.at[pl.ds(1 + 128 * k, 128)],
                sem,
            )
            for k in range(5)
        ]
        for c in copies:
            c.start()
        for c in copies:
            c.wait()

        for j in range(_D // _L):
            block[0, pl.ds(j * _L, _L)] = clsv[pl.ds(j * _L, _L)]

        def per_t(t, _):
            xnum = xv[pl.ds(t * _F, _L)]
            for f in range(_NUM_NUMERICAL):
                r = 1 + t * _F + f
                xs = jnp.full((_L,), xnum[f], jnp.float32)
                for j in range(_D // _L):
                    s = pl.ds(j * _L, _L)
                    block[r, s] = xs * wv[f, s] + (bv[f, s] + posv[t, s])

            def per_cat(c, _):
                r = 1 + t * _F + _NUM_NUMERICAL + c
                for j in range(_D // _L):
                    s = pl.ds(j * _L, _L)
                    block[r, s] = block[r, s] + posv[t, s]
                return _
            lax.fori_loop(0, _N_CAT, per_cat, None)
            return _
        lax.fori_loop(0, _T, per_t, None)

        pltpu.sync_copy(block, out_hbm.at[pl.ds(b * _ROWS, _ROWS)])
        return _

    lax.fori_loop(0, b_per_w, per_element, None)


def kernel(x_seq, cls_token, num_weights, num_biases, cat_tables, temporal_pos):
    B, T, F = x_seq.shape
    b_per_w = B // _NW

    x_flat = x_seq.reshape(B * T * F)
    table = cat_tables.reshape(_N_CAT * _VOCAB, _D)
    cls = cls_token.reshape(_D)

    mesh = plsc.VectorSubcoreMesh(core_axis_name="c", subcore_axis_name="s")
    out = pl.kernel(
        functools.partial(_tokenizer_body, b_per_w=b_per_w),
        out_type=jax.ShapeDtypeStruct((B * _ROWS, _D), jnp.float32),
        mesh=mesh,
        compiler_params=pltpu.CompilerParams(use_tc_tiling_on_sc=False),
        scratch_types=[
            pltpu.VMEM((_T * _F,), jnp.float32),
            pltpu.VMEM((5, 128), jnp.int32),
            pltpu.VMEM((_ROWS, _D), jnp.float32),
            pltpu.VMEM((_T, _D), jnp.float32),
            pltpu.VMEM((_NUM_NUMERICAL, _D), jnp.float32),
            pltpu.VMEM((_NUM_NUMERICAL, _D), jnp.float32),
            pltpu.VMEM((_D,), jnp.float32),
            pltpu.SemaphoreType.DMA,
        ],
    )(x_flat, cls, num_weights, num_biases, table, temporal_pos)
    return out.reshape(B, _ROWS, _D)


# speedup vs baseline: 1.1515x; 1.1515x over previous
import functools

import jax
import jax.numpy as jnp
from jax import lax
from jax.experimental import pallas as pl
from jax.experimental.pallas import tpu as pltpu
from jax.experimental.pallas import tpu_sc as plsc

_NUM_NUMERICAL = 6
_N_CAT = 26
_VOCAB = 100000
_D = 64
_T = 20
_F = _NUM_NUMERICAL + _N_CAT
_ROWS = 1 + _T * _F
_NC, _NS, _L = 2, 16, 16
_NW = _NC * _NS
_BB = 128
_G = _BB // _L


def _i16(val):
    return jnp.full((_L,), val, jnp.int32)


def _tokenizer_body(x_hbm, cls_hbm, w_hbm, b_hbm, table_hbm, pos_hbm, out_hbm,
                    xslab, idx3, half3, pairbuf3, outbufs, posv, wv, bv, clsv,
                    sg, so):
    wid = lax.axis_index("s") * _NC + lax.axis_index("c")
    b0 = wid * _BB

    pltpu.sync_copy(pos_hbm, posv)
    pltpu.sync_copy(w_hbm, wv)
    pltpu.sync_copy(b_hbm, bv)
    pltpu.sync_copy(cls_hbm, clsv)

    lane = lax.iota(jnp.int32, _L)
    rowidx = [lane + g * _L for g in range(_G)]

    def cls_d(d, _):
        csplat = plsc.load_gather(clsv, [_i16(d)])
        for g in range(_G):
            outbufs[0, d, pl.ds(g * _L, _L)] = csplat
        return _
    lax.fori_loop(0, _D, cls_d, None)
    pltpu.sync_copy(outbufs.at[0], out_hbm.at[0, :, pl.ds(b0, _BB)])

    def fire(c, jj):
        for g in range(_G):
            raw = xslab[_NUM_NUMERICAL + c, pl.ds(g * _L, _L)]
            vi = jnp.clip(raw.astype(jnp.int32), 0, _VOCAB - 1)
            idx3[jj, pl.ds(g * _L, _L)] = (
                c * (_VOCAB // 2) + lax.shift_right_logical(vi, 1))
            half3[jj, pl.ds(g * _L, _L)] = (vi & 1) * _D
        pltpu.make_async_copy(
            table_hbm.at[idx3.at[jj]], pairbuf3.at[jj], sg.at[jj]).start()

    def out_tile_write(p, t, f, first):
        r = 1 + t * _F + f
        dst = out_hbm.at[r, :, pl.ds(b0, _BB)]
        cp = pltpu.make_async_copy(outbufs.at[p], dst, so.at[p])

        @pl.when(jnp.logical_not(first))
        def _():
            cp.wait()
        return cp

    def per_t(t, _):
        pltpu.sync_copy(x_hbm.at[t, :, pl.ds(b0, _BB)], xslab)

        fire(0, 0)
        fire(1, 1)
        fire(2, 2)

        def per_cat(c, _):
            jj = c % 3
            p = c % 2
            pltpu.make_async_copy(
                table_hbm.at[idx3.at[jj]], pairbuf3.at[jj], sg.at[jj]).wait()

            first = jnp.logical_and(t == 0, c < 2)
            cp = out_tile_write(p, t, _NUM_NUMERICAL + c, first)

            halfv = [half3[jj, pl.ds(g * _L, _L)] for g in range(_G)]

            def per_d(d, _):
                psplat = plsc.load_gather(posv, [_i16(t), _i16(d)])
                for g in range(_G):
                    col = halfv[g] + d
                    vals = plsc.load_gather(
                        pairbuf3, [_i16(jj), rowidx[g], col])
                    outbufs[p, d, pl.ds(g * _L, _L)] = vals + psplat
                return _
            lax.fori_loop(0, _D, per_d, None)

            cp.start()

            @pl.when(c < _N_CAT - 3)
            def _():
                fire(c + 3, (c + 3) % 3)
            return _
        lax.fori_loop(0, _N_CAT, per_cat, None)

        def per_num(f, _):
            p = (_N_CAT + f) % 2
            cp = out_tile_write(p, t, f, jnp.bool_(False))
            xvecs = [xslab[f, pl.ds(g * _L, _L)] for g in range(_G)]

            def per_d(d, _):
                wsplat = plsc.load_gather(wv, [_i16(f), _i16(d)])
                bsplat = plsc.load_gather(bv, [_i16(f), _i16(d)])
                psplat = plsc.load_gather(posv, [_i16(t), _i16(d)])
                bp = bsplat + psplat
                for g in range(_G):
                    outbufs[p, d, pl.ds(g * _L, _L)] = xvecs[g] * wsplat + bp
                return _
            lax.fori_loop(0, _D, per_d, None)

            cp.start()
            return _
        lax.fori_loop(0, _NUM_NUMERICAL, per_num, None)
        return _

    lax.fori_loop(0, _T, per_t, None)

    for p in range(2):
        pltpu.make_async_copy(
            outbufs.at[p], out_hbm.at[0, :, pl.ds(b0, _BB)], so.at[p]).wait()


def kernel(x_seq, cls_token, num_weights, num_biases, cat_tables, temporal_pos):
    B, T, F = x_seq.shape

    x_t = jnp.transpose(x_seq, (1, 2, 0))
    table = cat_tables.reshape(_N_CAT * _VOCAB // 2, 2 * _D)
    cls = cls_token.reshape(_D)

    mesh = plsc.VectorSubcoreMesh(core_axis_name="c", subcore_axis_name="s")
    out_t = pl.kernel(
        _tokenizer_body,
        out_type=jax.ShapeDtypeStruct((_ROWS, _D, B), jnp.float32),
        mesh=mesh,
        compiler_params=pltpu.CompilerParams(
            use_tc_tiling_on_sc=True, needs_layout_passes=False),
        scratch_types=[
            pltpu.VMEM((_F, _BB), jnp.float32),
            pltpu.VMEM((3, _BB), jnp.int32),
            pltpu.VMEM((3, _BB), jnp.int32),
            pltpu.VMEM((3, _BB, 2 * _D), jnp.float32),
            pltpu.VMEM((2, _D, _BB), jnp.float32),
            pltpu.VMEM((_T, _D), jnp.float32),
            pltpu.VMEM((_NUM_NUMERICAL, _D), jnp.float32),
            pltpu.VMEM((_NUM_NUMERICAL, _D), jnp.float32),
            pltpu.VMEM((_D,), jnp.float32),
            pltpu.SemaphoreType.DMA((3,)),
            pltpu.SemaphoreType.DMA((2,)),
        ],
    )(x_t, cls, num_weights, num_biases, table, temporal_pos)
    return jnp.transpose(out_t, (2, 0, 1))
